# baseline (speedup 1.0000x reference)
import functools

import jax
import jax.numpy as jnp
from jax import lax
from jax.experimental import pallas as pl
from jax.experimental.pallas import tpu as pltpu
from jax.experimental.pallas import tpu_sc as plsc

N = 10000
E = 320000
D = 128
H = 8
F = 16
HF = H * F
SLOPE = 0.2

NC = 2
NS = 16
NW = NC * NS
EPT = 10240
NPAD = 10240
RPT = NPAD // NS
PADROW = N + 1

SK = 64
SCH = EPT // SK
AK = 128
ACH = EPT // AK

_f32 = jnp.float32


def _lin_body(x_ref, w_ref, b_ref, abig_ref, h_ref, stab_ref):
    h = jnp.dot(x_ref[...], w_ref[...], preferred_element_type=_f32) + b_ref[...]
    h_ref[...] = h
    stab_ref[...] = jnp.dot(h, abig_ref[...], preferred_element_type=_f32)


def _linear(x, w, b2, abig):
    blk = 1024
    return pl.pallas_call(
        _lin_body,
        grid=(NPAD // blk,),
        in_specs=[
            pl.BlockSpec((blk, D), lambda i: (i, 0)),
            pl.BlockSpec((D, HF), lambda i: (0, 0)),
            pl.BlockSpec((1, HF), lambda i: (0, 0)),
            pl.BlockSpec((HF, 128), lambda i: (0, 0)),
        ],
        out_specs=[
            pl.BlockSpec((blk, HF), lambda i: (i, 0)),
            pl.BlockSpec((blk, 128), lambda i: (i, 0)),
        ],
        out_shape=[
            jax.ShapeDtypeStruct((NPAD, HF), _f32),
            jax.ShapeDtypeStruct((NPAD, 128), _f32),
        ],
    )(x, w, b2, abig)


def _mesh():
    return plsc.VectorSubcoreMesh(core_axis_name="c", subcore_axis_name="s")


def _zero_shared_slice(zb, k, shared, sid):
    def zrow(i, _):
        for j in range(8):
            zb[i, pl.ds(16 * j, 16)] = jnp.zeros((16,), _f32)
        return 0
    lax.fori_loop(0, k, zrow, 0)
    for i in range(RPT // k):
        pltpu.sync_copy(zb, shared.at[pl.ds(sid * RPT + i * k, k)])


def _scores_body(tgt_h, src_h, stab, ex_out, den_out,
                 tgA, sgA, tgB, sgB, trA, srA, trB, srB, exA, exB,
                 den_sh, gA1, gA2, gB1, gB2, sA, sB, eA, eB):
    cid = lax.axis_index("c")
    sid = lax.axis_index("s")
    wid = cid * NS + sid

    _zero_shared_slice(trA, SK, den_sh, sid)
    plsc.subcore_barrier()

    lane = lax.iota(jnp.int32, 16)

    def gather(c, tg, sg, tr, sr, g1, g2):
        base = wid * EPT + c * SK
        pltpu.sync_copy(tgt_h.at[pl.ds(base, SK)], tg)
        pltpu.sync_copy(src_h.at[pl.ds(base, SK)], sg)
        cp1 = pltpu.async_copy(stab.at[tg], tr, g1)
        cp2 = pltpu.async_copy(stab.at[sg], sr, g2)
        return cp1, cp2

    lo8 = lane < 8
    tail = jnp.where(lane == 8, jnp.ones((16,), _f32),
                     jnp.zeros((16,), _f32))

    def compute(tr, sr, exf):
        def edge4(e4, _):
            for k in range(4):
                e = e4 * 4 + k
                v = tr[e, pl.ds(0, 16)] + sr[e, pl.ds(16, 16)]
                v = jnp.where(v > 0, v, SLOPE * v)
                v = jnp.exp(v)
                v = jnp.where(lo8, v, tail)
                exf[pl.ds(e * 16, 16)] = v
                tr[e, pl.ds(0, 16)] = v
            return 0
        lax.fori_loop(0, SK // 4, edge4, 0)

    def scat(c, tg, tr, exf, ssem, esem):
        cp1 = pltpu.async_copy(tr, den_sh.at[tg], ssem, add=True)
        base = wid * EPT + c * SK
        cp2 = pltpu.async_copy(exf, ex_out.at[pl.ds(base * 16, SK * 16)], esem)
        return cp1, cp2

    def pair(p, _):
        c0 = 2 * p
        c1 = c0 + 1
        ga = gather(c0, tgA, sgA, trA, srA, gA1, gA2)
        gb = gather(c1, tgB, sgB, trB, srB, gB1, gB2)
        ga[0].wait()
        ga[1].wait()
        compute(trA, srA, exA)
        sa = scat(c0, tgA, trA, exA, sA, eA)
        gb[0].wait()
        gb[1].wait()
        compute(trB, srB, exB)
        sb = scat(c1, tgB, trB, exB, sB, eB)
        sa[0].wait()
        sa[1].wait()
        sb[0].wait()
        sb[1].wait()
        return 0

    lax.fori_loop(0, SCH // 2, pair, 0)

    plsc.subcore_barrier()
    for i in range(RPT // 128):
        pltpu.sync_copy(den_sh.at[pl.ds(sid * RPT + i * 128, 128)],
                        den_out.at[cid, pl.ds(sid * RPT + i * 128, 128)])


def _scores(tgt, src, stab):
    k = functools.partial(
        pl.kernel,
        mesh=_mesh(),
        out_type=[
            jax.ShapeDtypeStruct((NW * EPT * 16,), _f32),
            jax.ShapeDtypeStruct((NC, NPAD, 128), _f32),
        ],
        scratch_types=[
            pltpu.VMEM((SK,), jnp.int32),
            pltpu.VMEM((SK,), jnp.int32),
            pltpu.VMEM((SK,), jnp.int32),
            pltpu.VMEM((SK,), jnp.int32),
            pltpu.VMEM((SK, 128), _f32),
            pltpu.VMEM((SK, 128), _f32),
            pltpu.VMEM((SK, 128), _f32),
            pltpu.VMEM((SK, 128), _f32),
            pltpu.VMEM((SK * 16,), _f32),
            pltpu.VMEM((SK * 16,), _f32),
            pltpu.VMEM_SHARED((NPAD, 128), _f32),
        ] + [pltpu.SemaphoreType.DMA] * 8,
    )(_scores_body)
    return k(tgt, src, stab)


def _agg_body(tgt_h, src_h, h, ex, out_hbm,
              tgA, sgA, tgB, sgB, hrA, hrB, evA, evB,
              out_sh, gA, gB, xA, xB, sA, sB):
    cid = lax.axis_index("c")
    sid = lax.axis_index("s")
    wid = cid * NS + sid

    _zero_shared_slice(hrA, AK, out_sh, sid)
    plsc.subcore_barrier()

    def gather(c, tg, sg, hr, ev, g, x):
        base = wid * EPT + c * AK
        pltpu.sync_copy(tgt_h.at[pl.ds(base, AK)], tg)
        pltpu.sync_copy(src_h.at[pl.ds(base, AK)], sg)
        cp1 = pltpu.async_copy(h.at[sg], hr, g)
        cp2 = pltpu.async_copy(ex.at[pl.ds(base * 16, AK * 16)], ev, x)
        return cp1, cp2

    jidx = [jnp.full((16,), j, jnp.int32) for j in range(8)]

    def compute(hr, ev):
        def edge4(e4, _):
            for k in range(4):
                e = e4 * 4 + k
                alpha = ev[pl.ds(e * 16, 16)]
                for j in range(8):
                    bc = alpha.at[jidx[j]].get(mode="promise_in_bounds")
                    hr[e, pl.ds(16 * j, 16)] = hr[e, pl.ds(16 * j, 16)] * bc
            return 0
        lax.fori_loop(0, AK // 4, edge4, 0)

    def pair(p, _):
        c0 = 2 * p
        c1 = c0 + 1
        ga = gather(c0, tgA, sgA, hrA, evA, gA, xA)
        gb = gather(c1, tgB, sgB, hrB, evB, gB, xB)
        ga[0].wait()
        ga[1].wait()
        compute(hrA, evA)
        sa = pltpu.async_copy(hrA, out_sh.at[tgA], sA, add=True)
        gb[0].wait()
        gb[1].wait()
        compute(hrB, evB)
        sb = pltpu.async_copy(hrB, out_sh.at[tgB], sB, add=True)
        sa.wait()
        sb.wait()
        return 0

    lax.fori_loop(0, ACH // 2, pair, 0)

    plsc.subcore_barrier()
    for i in range(RPT // 128):
        pltpu.sync_copy(out_sh.at[pl.ds(sid * RPT + i * 128, 128)],
                        out_hbm.at[cid, pl.ds(sid * RPT + i * 128, 128)])


def _aggregate(tgt, src, h, ex):
    k = functools.partial(
        pl.kernel,
        mesh=_mesh(),
        out_type=jax.ShapeDtypeStruct((NC, NPAD, HF), _f32),
        scratch_types=[
            pltpu.VMEM((AK,), jnp.int32),
            pltpu.VMEM((AK,), jnp.int32),
            pltpu.VMEM((AK,), jnp.int32),
            pltpu.VMEM((AK,), jnp.int32),
            pltpu.VMEM((AK, HF), _f32),
            pltpu.VMEM((AK, HF), _f32),
            pltpu.VMEM((AK * 16,), _f32),
            pltpu.VMEM((AK * 16,), _f32),
            pltpu.VMEM_SHARED((NPAD, HF), _f32),
        ] + [pltpu.SemaphoreType.DMA] * 6,
    )(_agg_body)
    return k(tgt, src, h, ex)


def _fin_body(p0_ref, p1_ref, d0_ref, d1_ref, h_ref, bb_ref, bd_ref, o_ref):
    d = d0_ref[...] + d1_ref[...]
    lane = lax.broadcasted_iota(jnp.int32, d.shape, 1)
    rinv = jnp.where(lane < 8, 1.0 / (d + 1e-16), 0.0)
    degc = jnp.where(lane == 8, d, 0.0)
    rb = jnp.dot(rinv, bb_ref[...], preferred_element_type=_f32)
    db = jnp.dot(degc, bd_ref[...], preferred_element_type=_f32)
    y = (p0_ref[...] + p1_ref[...]) * rb + db * h_ref[...]
    o_ref[...] = jnp.where(y > 0, y, jnp.exp(y) - 1.0)


def _final(p0, p1, d0, d1, h, bb, bd):
    blk = 1000
    return pl.pallas_call(
        _fin_body,
        grid=(N // blk,),
        in_specs=[
            pl.BlockSpec((blk, HF), lambda i: (i, 0)),
            pl.BlockSpec((blk, HF), lambda i: (i, 0)),
            pl.BlockSpec((blk, 128), lambda i: (i, 0)),
            pl.BlockSpec((blk, 128), lambda i: (i, 0)),
            pl.BlockSpec((blk, HF), lambda i: (i, 0)),
            pl.BlockSpec((128, 128), lambda i: (0, 0)),
            pl.BlockSpec((128, 128), lambda i: (0, 0)),
        ],
        out_specs=pl.BlockSpec((blk, HF), lambda i: (i, 0)),
        out_shape=jax.ShapeDtypeStruct((N, HF), _f32),
    )(p0, p1, d0, d1, h, bb, bd)


def kernel(node_features, edge_index, W, b, a):
    f_idx = jnp.arange(HF) % F
    h_idx = jnp.arange(HF) // F
    oh = jax.nn.one_hot(h_idx, H, dtype=_f32)
    a1 = oh * a[:F][f_idx][:, None]
    a2 = oh * a[F:][f_idx][:, None]
    abig = jnp.concatenate(
        [a1, a2, a2, a1, jnp.zeros((HF, 96), _f32)], axis=1)

    li = jnp.arange(128)
    bb = jnp.where((li[:, None] < 8) & ((li[None, :] // 16) == li[:, None]),
                   1.0, 0.0).astype(_f32)
    bd = jnp.where(li[:, None] == 8, 1.0, 0.0).astype(_f32)

    real = E // NW
    tgt2 = edge_index[1].reshape(NW, real)
    src2 = edge_index[0].reshape(NW, real)
    padt = jnp.full((NW, EPT - real), PADROW, jnp.int32)
    pads = jnp.full((NW, EPT - real), N, jnp.int32)
    tgtf = jnp.concatenate([tgt2, padt], axis=1).reshape(NW * EPT)
    srcf = jnp.concatenate([src2, pads], axis=1).reshape(NW * EPT)
    xp = jnp.zeros((NPAD, D), _f32).at[0:N].set(node_features)

    h, stab = _linear(xp, W, b.reshape(1, HF), abig)
    ex, den = _scores(tgtf, srcf, stab)
    part = _aggregate(tgtf, srcf, h, ex)
    return _final(part[0], part[1], den[0, 0:N], den[1, 0:N], h[0:N], bb, bd)


# speedup vs baseline: 1.4312x; 1.4312x over previous
import functools

import jax
import jax.numpy as jnp
from jax import lax
from jax.experimental import pallas as pl
from jax.experimental.pallas import tpu as pltpu
from jax.experimental.pallas import tpu_sc as plsc

N = 10000
E = 320000
D = 128
H = 8
F = 16
HF = H * F
SLOPE = 0.2

NC = 2
NS = 16
NW = NC * NS
EPT = E // NW
K = 80
CHUNKS = EPT // K
NPAD = 10240
RPT = NPAD // NS

_f32 = jnp.float32


def _lin_body(x_ref, w_ref, b_ref, abig_ref, h_ref, stab_ref):
    h = jnp.dot(x_ref[...], w_ref[...], preferred_element_type=_f32) + b_ref[...]
    h_ref[...] = h
    stab_ref[...] = jnp.dot(h, abig_ref[...], preferred_element_type=_f32)


def _linear(x, w, b2, abig):
    blk = 1000
    return pl.pallas_call(
        _lin_body,
        grid=(N // blk,),
        in_specs=[
            pl.BlockSpec((blk, D), lambda i: (i, 0)),
            pl.BlockSpec((D, HF), lambda i: (0, 0)),
            pl.BlockSpec((1, HF), lambda i: (0, 0)),
            pl.BlockSpec((HF, 128), lambda i: (0, 0)),
        ],
        out_specs=[
            pl.BlockSpec((blk, HF), lambda i: (i, 0)),
            pl.BlockSpec((blk, 128), lambda i: (i, 0)),
        ],
        out_shape=[
            jax.ShapeDtypeStruct((N, HF), _f32),
            jax.ShapeDtypeStruct((N, 128), _f32),
        ],
    )(x, w, b2, abig)


def _mesh():
    return plsc.VectorSubcoreMesh(core_axis_name="c", subcore_axis_name="s")


def _zero_shared_slice(zbuf, shared, sid):
    def zrow(i, _):
        for j in range(8):
            zbuf[i, pl.ds(16 * j, 16)] = jnp.zeros((16,), _f32)
        return 0
    lax.fori_loop(0, 128, zrow, 0)
    for i in range(RPT // 128):
        pltpu.sync_copy(zbuf, shared.at[pl.ds(sid * RPT + i * 128, 128)])


def _scores_body(src_h, tgt_h, stab, ex_out, den_out,
                 tgt_v, src_v, trows, srows, exst, exst128, zbuf, den_sh,
                 sem1, sem2, sem3, sem4):
    cid = lax.axis_index("c")
    sid = lax.axis_index("s")
    wid = cid * NS + sid

    _zero_shared_slice(zbuf, den_sh, sid)
    def zpad(e, _):
        for j in range(1, 8):
            exst128[e, pl.ds(16 * j, 16)] = jnp.zeros((16,), _f32)
        return 0
    lax.fori_loop(0, K, zpad, 0)
    plsc.subcore_barrier()

    lane = lax.iota(jnp.int32, 16)

    def chunk(c, _):
        base = wid * EPT + c * K
        ci1 = pltpu.async_copy(tgt_h.at[pl.ds(base, K)], tgt_v, sem3)
        ci2 = pltpu.async_copy(src_h.at[pl.ds(base, K)], src_v, sem4)
        ci1.wait()
        ci2.wait()
        cp1 = pltpu.async_copy(stab.at[tgt_v], trows, sem1)
        cp2 = pltpu.async_copy(stab.at[src_v], srows, sem2)
        cp1.wait()
        cp2.wait()

        def edge(e, _):
            v = trows[e, pl.ds(0, 16)] + srows[e, pl.ds(16, 16)]
            v = jnp.where(v > 0, v, SLOPE * v)
            v = jnp.exp(v)
            v = jnp.where(lane < 8, v,
                          jnp.where(lane == 8, jnp.ones((16,), _f32),
                                    jnp.zeros((16,), _f32)))
            exst[pl.ds(e * 16, 16)] = v
            exst128[e, pl.ds(0, 16)] = v
            return 0
        lax.fori_loop(0, K, edge, 0)

        ce = pltpu.async_copy(exst, ex_out.at[pl.ds(base * 16, K * 16)], sem3)
        cs = pltpu.async_copy(exst128, den_sh.at[tgt_v], sem4, add=True)
        ce.wait()
        cs.wait()
        return 0

    lax.fori_loop(0, CHUNKS, chunk, 0)
    plsc.subcore_barrier()
    for i in range(RPT // 128):
        pltpu.sync_copy(den_sh.at[pl.ds(sid * RPT + i * 128, 128)],
                        den_out.at[cid, pl.ds(sid * RPT + i * 128, 128)])


def _scores(src, tgt, stab):
    k = functools.partial(
        pl.kernel,
        mesh=_mesh(),
        out_type=[
            jax.ShapeDtypeStruct((E * 16,), _f32),
            jax.ShapeDtypeStruct((NC, NPAD, 128), _f32),
        ],
        scratch_types=[
            pltpu.VMEM((K,), jnp.int32),
            pltpu.VMEM((K,), jnp.int32),
            pltpu.VMEM((K, 128), _f32),
            pltpu.VMEM((K, 128), _f32),
            pltpu.VMEM((K * 16,), _f32),
            pltpu.VMEM((K, 128), _f32),
            pltpu.VMEM((128, 128), _f32),
            pltpu.VMEM_SHARED((NPAD, 128), _f32),
            pltpu.SemaphoreType.DMA,
            pltpu.SemaphoreType.DMA,
            pltpu.SemaphoreType.DMA,
            pltpu.SemaphoreType.DMA,
        ],
    )(_scores_body)
    return k(src, tgt, stab)


def _agg_body(src_h, tgt_h, h, ex, out_hbm,
              tgt_v, src_v, hrows, ex_v, zbuf, out_sh,
              sem1, sem2, sem3, sem4):
    cid = lax.axis_index("c")
    sid = lax.axis_index("s")
    wid = cid * NS + sid

    _zero_shared_slice(zbuf, out_sh, sid)
    plsc.subcore_barrier()

    def chunk(c, _):
        base = wid * EPT + c * K
        ci1 = pltpu.async_copy(tgt_h.at[pl.ds(base, K)], tgt_v, sem2)
        ci2 = pltpu.async_copy(src_h.at[pl.ds(base, K)], src_v, sem3)
        ci2.wait()
        cp1 = pltpu.async_copy(h.at[src_v], hrows, sem1)
        cp2 = pltpu.async_copy(ex.at[pl.ds(base * 16, K * 16)], ex_v, sem4)
        ci1.wait()
        cp2.wait()
        cp1.wait()

        def edge(e, _):
            alpha = ex_v[pl.ds(e * 16, 16)]
            for j in range(8):
                a_s = alpha[j]
                hrows[e, pl.ds(16 * j, 16)] = hrows[e, pl.ds(16 * j, 16)] * a_s
            return 0
        lax.fori_loop(0, K, edge, 0)

        pltpu.sync_copy(hrows, out_sh.at[tgt_v], add=True)
        return 0

    lax.fori_loop(0, CHUNKS, chunk, 0)
    plsc.subcore_barrier()
    for i in range(RPT // 128):
        pltpu.sync_copy(out_sh.at[pl.ds(sid * RPT + i * 128, 128)],
                        out_hbm.at[cid, pl.ds(sid * RPT + i * 128, 128)])


def _aggregate(src, tgt, h, ex):
    k = functools.partial(
        pl.kernel,
        mesh=_mesh(),
        out_type=jax.ShapeDtypeStruct((NC, NPAD, HF), _f32),
        scratch_types=[
            pltpu.VMEM((K,), jnp.int32),
            pltpu.VMEM((K,), jnp.int32),
            pltpu.VMEM((K, HF), _f32),
            pltpu.VMEM((K * 16,), _f32),
            pltpu.VMEM((128, HF), _f32),
            pltpu.VMEM_SHARED((NPAD, HF), _f32),
            pltpu.SemaphoreType.DMA,
            pltpu.SemaphoreType.DMA,
            pltpu.SemaphoreType.DMA,
            pltpu.SemaphoreType.DMA,
        ],
    )(_agg_body)
    return k(src, tgt, h, ex)


def _fin_body(p0_ref, p1_ref, d0_ref, d1_ref, h_ref, bb_ref, bd_ref, o_ref):
    d = d0_ref[...] + d1_ref[...]
    lane = lax.broadcasted_iota(jnp.int32, d.shape, 1)
    rinv = jnp.where(lane < 8, 1.0 / (d + 1e-16), 0.0)
    degc = jnp.where(lane == 8, d, 0.0)
    rb = jnp.dot(rinv, bb_ref[...], preferred_element_type=_f32)
    db = jnp.dot(degc, bd_ref[...], preferred_element_type=_f32)
    y = (p0_ref[...] + p1_ref[...]) * rb + db * h_ref[...]
    o_ref[...] = jnp.where(y > 0, y, jnp.exp(y) - 1.0)


def _final(p0, p1, d0, d1, h, bb, bd):
    blk = 1000
    return pl.pallas_call(
        _fin_body,
        grid=(N // blk,),
        in_specs=[
            pl.BlockSpec((blk, HF), lambda i: (i, 0)),
            pl.BlockSpec((blk, HF), lambda i: (i, 0)),
            pl.BlockSpec((blk, 128), lambda i: (i, 0)),
            pl.BlockSpec((blk, 128), lambda i: (i, 0)),
            pl.BlockSpec((blk, HF), lambda i: (i, 0)),
            pl.BlockSpec((128, 128), lambda i: (0, 0)),
            pl.BlockSpec((128, 128), lambda i: (0, 0)),
        ],
        out_specs=pl.BlockSpec((blk, HF), lambda i: (i, 0)),
        out_shape=jax.ShapeDtypeStruct((N, HF), _f32),
    )(p0, p1, d0, d1, h, bb, bd)


def kernel(node_features, edge_index, W, b, a):
    f_idx = jnp.arange(HF) % F
    h_idx = jnp.arange(HF) // F
    oh = jax.nn.one_hot(h_idx, H, dtype=_f32)
    a1 = oh * a[:F][f_idx][:, None]
    a2 = oh * a[F:][f_idx][:, None]
    abig = jnp.concatenate(
        [a1, a2, a2, a1, jnp.zeros((HF, 96), _f32)], axis=1)

    li = jnp.arange(128)
    bb = jnp.where((jnp.arange(128)[:, None] < 8)
                   & ((li[None, :] // 16) == jnp.arange(128)[:, None]),
                   1.0, 0.0).astype(_f32)
    bd = jnp.where(jnp.arange(128)[:, None] == 8, 1.0, 0.0).astype(_f32)

    src = edge_index[0]
    tgt = edge_index[1]
    h, stab = _linear(node_features, W, b.reshape(1, HF), abig)
    ex, den = _scores(src, tgt, stab)
    part = _aggregate(src, tgt, h, ex)
    return _final(part[0], part[1], den[0, 0:N], den[1, 0:N], h, bb, bd)
